# baseline (speedup 1.0000x reference)
import functools

import jax
import jax.numpy as jnp
from jax import lax
from jax.experimental import pallas as pl
from jax.experimental.pallas import tpu as pltpu
from jax.experimental.pallas import tpu_sc as plsc

N_USERS = 1000000
N_ITEMS = 1000000
EMB_DIM = 64
BATCH = 4096

_BN = 256


@functools.cache
def _make_sc_gather():
    info = plsc.get_sparse_core_info()
    nc, ns = info.num_cores, info.num_subcores
    bpw = BATCH // (nc * ns)

    def body(user_hbm, item_hbm, users_hbm, pos_hbm, u_out, i_out,
             uidx_v, iidx_v, urows_v, irows_v, sem_u, sem_i):
        wid = lax.axis_index("s") * nc + lax.axis_index("c")
        base = wid * bpw
        pltpu.sync_copy(users_hbm.at[pl.ds(base, bpw)], uidx_v)
        pltpu.sync_copy(pos_hbm.at[pl.ds(base, bpw)], iidx_v)
        cu = pltpu.async_copy(user_hbm.at[uidx_v], urows_v, sem_u)
        ci = pltpu.async_copy(item_hbm.at[iidx_v], irows_v, sem_i)
        cu.wait()
        ci.wait()
        pltpu.sync_copy(urows_v, u_out.at[pl.ds(base, bpw)])
        pltpu.sync_copy(irows_v, i_out.at[pl.ds(base, bpw)])

    return pl.kernel(
        body,
        mesh=plsc.VectorSubcoreMesh(core_axis_name="c", subcore_axis_name="s"),
        compiler_params=pltpu.CompilerParams(use_tc_tiling_on_sc=False),
        out_type=[
            jax.ShapeDtypeStruct((BATCH, EMB_DIM), jnp.float32),
            jax.ShapeDtypeStruct((BATCH, EMB_DIM), jnp.float32),
        ],
        scratch_types=[
            pltpu.VMEM((bpw,), jnp.int32),
            pltpu.VMEM((bpw,), jnp.int32),
            pltpu.VMEM((bpw, EMB_DIM), jnp.float32),
            pltpu.VMEM((bpw, EMB_DIM), jnp.float32),
            pltpu.SemaphoreType.DMA,
            pltpu.SemaphoreType.DMA,
        ],
    )


def _normalize_bf16(x):
    norm = jnp.sqrt(jnp.sum(x * x, axis=-1, keepdims=True))
    return (x / jnp.maximum(norm, 1e-12)).astype(jnp.bfloat16)


def _mm_body(u_ref, i_ref, o_ref, un_scratch):
    j = pl.program_id(0)

    @pl.when(j == 0)
    def _():
        un_scratch[...] = _normalize_bf16(u_ref[...])

    ib = _normalize_bf16(i_ref[...])
    o_ref[...] = lax.dot_general(
        un_scratch[...], ib,
        dimension_numbers=(((1,), (1,)), ((), ())),
        preferred_element_type=jnp.float32,
    )


def _tc_score(u_e, i_e):
    grid = (BATCH // _BN,)
    return pl.pallas_call(
        _mm_body,
        grid=grid,
        in_specs=[
            pl.BlockSpec((BATCH, EMB_DIM), lambda j: (0, 0)),
            pl.BlockSpec((_BN, EMB_DIM), lambda j: (j, 0)),
        ],
        out_specs=pl.BlockSpec((BATCH, _BN), lambda j: (0, j)),
        out_shape=jax.ShapeDtypeStruct((BATCH, BATCH), jnp.float32),
        scratch_shapes=[pltpu.VMEM((BATCH, EMB_DIM), jnp.bfloat16)],
    )(u_e, i_e)


def kernel(user_embedding, item_embedding, users, pos_items):
    users = users.astype(jnp.int32)
    pos_items = pos_items.astype(jnp.int32)
    u_e, i_e = _make_sc_gather()(user_embedding, item_embedding, users, pos_items)
    return _tc_score(u_e, i_e)
